# baseline (speedup 1.0000x reference)
import functools

import jax
import jax.numpy as jnp
from jax import lax
from jax.experimental import pallas as pl
from jax.experimental.pallas import tpu as pltpu
from jax.experimental.pallas import tpu_sc as plsc

_EMBED = 128
_MAXD2 = 256
_B = 1024 * 200
_NW = 32
_PER_W = _B // _NW
_CHUNK = 256
_NCHUNK = _PER_W // _CHUNK
_IDX_ROWS = _PER_W // _EMBED
_IPC = _CHUNK // _EMBED
_NBUF = 2
_G = 1

_mesh = plsc.VectorSubcoreMesh(core_axis_name="c", subcore_axis_name="s")


@functools.partial(
    pl.kernel,
    out_type=jax.ShapeDtypeStruct((_B, _EMBED), jnp.float32),
    mesh=_mesh,
    scratch_types=[
        pltpu.VMEM((_PER_W,), jnp.int32),
        pltpu.VMEM((_PER_W,), jnp.int32),
        pltpu.VMEM((_PER_W,), jnp.int32),
        pltpu.VMEM((_NBUF, _CHUNK, _EMBED), jnp.float32),
        pltpu.SemaphoreType.DMA((_NBUF,)),
        pltpu.SemaphoreType.DMA((_NBUF,)),
    ],
)
def _gather_kernel(d1_hbm, d2_hbm, table_hbm, out_hbm,
                   d1_v, d2_v, idx_v, rows_v, sem_g, sem_s):
    wid = lax.axis_index("s") * 2 + lax.axis_index("c")
    base = wid * _PER_W

    pltpu.sync_copy(d1_hbm.at[pl.ds(base, _PER_W)], d1_v)
    pltpu.sync_copy(d2_hbm.at[pl.ds(base, _PER_W)], d2_v)

    @pl.loop(0, _PER_W // 16)
    def _compute_idx(j):
        s = pl.ds(j * 16, 16)
        idx_v[s] = d1_v[s] * _MAXD2 + d2_v[s]

    for j in range(_G):
        pltpu.async_copy(table_hbm.at[idx_v.at[pl.ds(j * _CHUNK, _CHUNK)]], rows_v.at[j], sem_g.at[j])

    @pl.loop(0, _NCHUNK)
    def _chunk(j):
        b = lax.rem(j, _NBUF)

        @pl.when(j < _NCHUNK - _G)
        def _prefetch():
            nb = lax.rem(j + _G, _NBUF)

            @pl.when(j >= _NBUF - _G)
            def _wait_old_store():
                pltpu.make_async_copy(
                    rows_v.at[nb],
                    out_hbm.at[pl.ds(base, _CHUNK)],
                    sem_s.at[nb],
                ).wait()
            pltpu.async_copy(
                table_hbm.at[idx_v.at[pl.ds((j + _G) * _CHUNK, _CHUNK)]], rows_v.at[nb], sem_g.at[nb])

        pltpu.make_async_copy(
            table_hbm.at[idx_v.at[pl.ds(j * _CHUNK, _CHUNK)]], rows_v.at[b], sem_g.at[b]).wait()
        pltpu.async_copy(
            rows_v.at[b],
            out_hbm.at[pl.ds(base + j * _CHUNK, _CHUNK)],
            sem_s.at[b],
        )

    for t in range(_NCHUNK - _NBUF + _G, _NCHUNK):
        pltpu.make_async_copy(
            rows_v.at[t % _NBUF],
            out_hbm.at[pl.ds(base, _CHUNK)],
            sem_s.at[t % _NBUF],
        ).wait()


def kernel(dim1_indices, dim2_indices, pos_embed):
    d1 = dim1_indices.reshape(-1)
    d2 = dim2_indices.reshape(-1)
    out = _gather_kernel(d1, d2, pos_embed)
    return out.reshape(dim1_indices.shape + (pos_embed.shape[1],))


# speedup vs baseline: 1.0393x; 1.0393x over previous
import functools

import jax
import jax.numpy as jnp
from jax import lax
from jax.experimental import pallas as pl
from jax.experimental.pallas import tpu as pltpu
from jax.experimental.pallas import tpu_sc as plsc

_EMBED = 128
_MAXD2 = 256
_B = 1024 * 200
_NW = 32
_PER_W = _B // _NW
_CHUNK = 128
_NCHUNK = _PER_W // _CHUNK
_NBUF = 4
_G = 2

_mesh = plsc.VectorSubcoreMesh(core_axis_name="c", subcore_axis_name="s")


@functools.partial(
    pl.kernel,
    out_type=jax.ShapeDtypeStruct((_B, _EMBED), jnp.float32),
    mesh=_mesh,
    scratch_types=[
        pltpu.VMEM((_PER_W,), jnp.int32),
        pltpu.VMEM((_PER_W,), jnp.int32),
        pltpu.VMEM((_PER_W,), jnp.int32),
        pltpu.VMEM((_NBUF, _CHUNK, _EMBED), jnp.float32),
        pltpu.SemaphoreType.DMA((_NBUF,)),
        pltpu.SemaphoreType.DMA((_NBUF,)),
    ],
)
def _gather_kernel(d1_hbm, d2_hbm, table_hbm, out_hbm,
                   d1_v, d2_v, idx_v, rows_v, sem_g, sem_s):
    wid = lax.axis_index("s") * 2 + lax.axis_index("c")
    base = wid * _PER_W

    pltpu.sync_copy(d1_hbm.at[pl.ds(base, _PER_W)], d1_v)
    pltpu.sync_copy(d2_hbm.at[pl.ds(base, _PER_W)], d2_v)

    def compute_idx(c):
        for k in range(_CHUNK // 16):
            s = pl.ds(c * _CHUNK + k * 16, 16)
            idx_v[s] = d1_v[s] * _MAXD2 + d2_v[s]

    def gather(c, buf):
        pltpu.async_copy(
            table_hbm.at[idx_v.at[pl.ds(c * _CHUNK, _CHUNK)]],
            rows_v.at[buf], sem_g.at[buf])

    for j in range(_G):
        compute_idx(j)
        gather(j, j)

    @pl.loop(0, _NCHUNK)
    def _chunk(j):
        b = lax.rem(j, _NBUF)

        @pl.when(j < _NCHUNK - _G)
        def _prefetch():
            nb = lax.rem(j + _G, _NBUF)
            compute_idx(j + _G)

            @pl.when(j >= _NBUF - _G)
            def _wait_old_store():
                pltpu.make_async_copy(
                    rows_v.at[nb],
                    out_hbm.at[pl.ds(base, _CHUNK)],
                    sem_s.at[nb],
                ).wait()
            gather(j + _G, nb)

        pltpu.make_async_copy(
            table_hbm.at[idx_v.at[pl.ds(j * _CHUNK, _CHUNK)]],
            rows_v.at[b], sem_g.at[b]).wait()
        pltpu.async_copy(
            rows_v.at[b],
            out_hbm.at[pl.ds(base + j * _CHUNK, _CHUNK)],
            sem_s.at[b],
        )

    for t in range(_NCHUNK - _NBUF + _G, _NCHUNK):
        pltpu.make_async_copy(
            rows_v.at[t % _NBUF],
            out_hbm.at[pl.ds(base, _CHUNK)],
            sem_s.at[t % _NBUF],
        ).wait()


def kernel(dim1_indices, dim2_indices, pos_embed):
    d1 = dim1_indices.reshape(-1)
    d2 = dim2_indices.reshape(-1)
    out = _gather_kernel(d1, d2, pos_embed)
    return out.reshape(dim1_indices.shape + (pos_embed.shape[1],))


# speedup vs baseline: 1.0470x; 1.0074x over previous
import functools

import jax
import jax.numpy as jnp
from jax import lax
from jax.experimental import pallas as pl
from jax.experimental.pallas import tpu as pltpu
from jax.experimental.pallas import tpu_sc as plsc

_EMBED = 128
_MAXD2 = 256
_B = 1024 * 200
_NW = 32
_PER_W = _B // _NW
_CHUNK = 128
_NCHUNK = _PER_W // _CHUNK
_NBUF = 6
_G = 2

_mesh = plsc.VectorSubcoreMesh(core_axis_name="c", subcore_axis_name="s")


@functools.partial(
    pl.kernel,
    out_type=jax.ShapeDtypeStruct((_B, _EMBED), jnp.float32),
    mesh=_mesh,
    scratch_types=[
        pltpu.VMEM((_PER_W,), jnp.int32),
        pltpu.VMEM((_PER_W,), jnp.int32),
        pltpu.VMEM((_PER_W,), jnp.int32),
        pltpu.VMEM((_NBUF, _CHUNK, _EMBED), jnp.float32),
        pltpu.SemaphoreType.DMA((_NBUF,)),
        pltpu.SemaphoreType.DMA((_NBUF,)),
    ],
)
def _gather_kernel(d1_hbm, d2_hbm, table_hbm, out_hbm,
                   d1_v, d2_v, idx_v, rows_v, sem_g, sem_s):
    wid = lax.axis_index("s") * 2 + lax.axis_index("c")
    base = wid * _PER_W

    pltpu.sync_copy(d1_hbm.at[pl.ds(base, _PER_W)], d1_v)
    pltpu.sync_copy(d2_hbm.at[pl.ds(base, _PER_W)], d2_v)

    def compute_idx(c):
        for k in range(_CHUNK // 16):
            s = pl.ds(c * _CHUNK + k * 16, 16)
            idx_v[s] = d1_v[s] * _MAXD2 + d2_v[s]

    def gather(c, buf):
        pltpu.async_copy(
            table_hbm.at[idx_v.at[pl.ds(c * _CHUNK, _CHUNK)]],
            rows_v.at[buf], sem_g.at[buf])

    for j in range(_G):
        compute_idx(j)
        gather(j, j)

    @pl.loop(0, _NCHUNK)
    def _chunk(j):
        b = lax.rem(j, _NBUF)

        @pl.when(j < _NCHUNK - _G)
        def _prefetch():
            nb = lax.rem(j + _G, _NBUF)
            compute_idx(j + _G)

            @pl.when(j >= _NBUF - _G)
            def _wait_old_store():
                pltpu.make_async_copy(
                    rows_v.at[nb],
                    out_hbm.at[pl.ds(base, _CHUNK)],
                    sem_s.at[nb],
                ).wait()
            gather(j + _G, nb)

        pltpu.make_async_copy(
            table_hbm.at[idx_v.at[pl.ds(j * _CHUNK, _CHUNK)]],
            rows_v.at[b], sem_g.at[b]).wait()
        pltpu.async_copy(
            rows_v.at[b],
            out_hbm.at[pl.ds(base + j * _CHUNK, _CHUNK)],
            sem_s.at[b],
        )

    for t in range(_NCHUNK - _NBUF + _G, _NCHUNK):
        pltpu.make_async_copy(
            rows_v.at[t % _NBUF],
            out_hbm.at[pl.ds(base, _CHUNK)],
            sem_s.at[t % _NBUF],
        ).wait()


def kernel(dim1_indices, dim2_indices, pos_embed):
    d1 = dim1_indices.reshape(-1)
    d2 = dim2_indices.reshape(-1)
    out = _gather_kernel(d1, d2, pos_embed)
    return out.reshape(dim1_indices.shape + (pos_embed.shape[1],))
